# Initial kernel scaffold; baseline (speedup 1.0000x reference)
import jax
import jax.numpy as jnp
from jax.experimental import pallas as pl


def kernel(x, pert_idx, batch, G_coexpress, G_coexpress_weight, G_go, G_go_weight, params):
    raise NotImplementedError("write your pallas kernel here")


# speedup vs baseline: 11.8316x; 11.8316x over previous
import functools

import jax
import jax.numpy as jnp
from jax import lax
from jax.experimental import pallas as pl
from jax.experimental.pallas import tpu as pltpu
from jax.experimental.pallas import tpu_sc as plsc

NG = 10000
NP = 5000
H = 64
NGR = 16
ECO = 320000
EGO = 160000
EPS = 1e-5
NROWS = NGR * NG

NJ = 5
TJ = NG // NJ

F32 = jnp.float32


def _edge_partition(wid, nbatch_total):
    nb_lo = nbatch_total // 32
    extra = nbatch_total - nb_lo * 32
    nb = nb_lo + jnp.where(wid < extra, 1, 0)
    base = nb_lo * wid + jnp.minimum(wid, extra)
    return nb, base


def _sc_deg(dst_g, ew_g, dst_p, ew_p, zrows):
    mesh = plsc.VectorSubcoreMesh(core_axis_name="c", subcore_axis_name="s")

    @functools.partial(
        pl.kernel, mesh=mesh,
        out_type=(jax.ShapeDtypeStruct((2 * NG, 128), F32),
                  jax.ShapeDtypeStruct((2 * NP, 128), F32)),
        scratch_types=(pltpu.VMEM((1, 64), jnp.int32),
                       pltpu.VMEM((1, 64), F32),
                       pltpu.VMEM((64, 128), F32),
                       pltpu.VMEM_SHARED((NG, 128), F32),
                       pltpu.VMEM_SHARED((NP, 128), F32),
                       pltpu.SemaphoreType.DMA),
    )
    def k(dstg_h, ewg_h, dstp_h, ewp_h, z_h, outg_h, outp_h,
          idxb, ewb, valb, degg_sh, degp_sh, sem):
        c = lax.axis_index("c")
        s = lax.axis_index("s")
        wid = c * 16 + s
        pltpu.sync_copy(z_h.at[pl.ds(s * 624, 624)], degg_sh.at[pl.ds(s * 624, 624)])
        pltpu.sync_copy(z_h.at[pl.ds(s * 312, 312)], degp_sh.at[pl.ds(s * 312, 312)])
        pltpu.sync_copy(z_h.at[pl.ds(0, 64)], valb)
        @pl.when(s == 15)
        def _():
            pltpu.sync_copy(z_h.at[pl.ds(9984, 16)], degg_sh.at[pl.ds(9984, 16)])
            pltpu.sync_copy(z_h.at[pl.ds(4992, 8)], degp_sh.at[pl.ds(4992, 8)])
        plsc.subcore_barrier()

        lane0 = lax.iota(jnp.int32, 16) == 0

        def scatter_edges(dst_h, ew_h, table_sh, nbatch_total):
            nb, base = _edge_partition(wid, nbatch_total)

            def body(b, carry):
                off = (base + b) * 64
                pltpu.sync_copy(dst_h.at[pl.ds(off, 64)], idxb.at[0])
                pltpu.sync_copy(ew_h.at[pl.ds(off, 64)], ewb.at[0])

                def vbody(gi, cc):
                    ev = ewb[0, pl.ds(gi * 16, 16)]
                    for kk in range(16):
                        e = gi * 16 + kk
                        valb[e, pl.ds(0, 16)] = jnp.where(lane0, ev[kk], 0.0)
                    return cc

                lax.fori_loop(0, 4, vbody, 0)
                pltpu.sync_copy(valb, table_sh.at[idxb.at[0]], add=True)
                return carry

            lax.fori_loop(0, nb, body, 0)

        scatter_edges(dstg_h, ewg_h, degg_sh, ECO // 64)
        scatter_edges(dstp_h, ewp_h, degp_sh, EGO // 64)
        plsc.subcore_barrier()

        pltpu.sync_copy(degg_sh.at[pl.ds(s * 624, 624)], outg_h.at[pl.ds(c * NG + s * 624, 624)])
        pltpu.sync_copy(degp_sh.at[pl.ds(s * 312, 312)], outp_h.at[pl.ds(c * NP + s * 312, 312)])
        @pl.when(s == 15)
        def _():
            pltpu.sync_copy(degg_sh.at[pl.ds(9984, 16)], outg_h.at[pl.ds(c * NG + 9984, 16)])
            pltpu.sync_copy(degp_sh.at[pl.ds(4992, 8)], outp_h.at[pl.ds(c * NP + 4992, 8)])

    return k(dst_g, ew_g, dst_p, ew_p, zrows)


def _sc_rows(src_g, dst_g, ew_g, pe2, src_p, dst_p, ew_p, pt2, zrows):
    mesh = plsc.VectorSubcoreMesh(core_axis_name="c", subcore_axis_name="s")

    @functools.partial(
        pl.kernel, mesh=mesh,
        out_type=(jax.ShapeDtypeStruct((2 * NG, 128), F32),
                  jax.ShapeDtypeStruct((2 * NP, 128), F32)),
        scratch_types=(pltpu.VMEM((1, 64), jnp.int32),
                       pltpu.VMEM((1, 64), jnp.int32),
                       pltpu.VMEM((1, 64), F32),
                       pltpu.VMEM((64, 128), F32),
                       pltpu.VMEM_SHARED((NG, 128), F32),
                       pltpu.VMEM_SHARED((NP, 128), F32),
                       pltpu.SemaphoreType.DMA),
    )
    def k(srcg_h, dstg_h, ewg_h, pe2_h, srcp_h, dstp_h, ewp_h, pt2_h, z_h,
          outg_h, outp_h, idxs, idxd, ewb, rowsb, rg_sh, rp_sh, sem):
        c = lax.axis_index("c")
        s = lax.axis_index("s")
        wid = c * 16 + s
        pltpu.sync_copy(z_h.at[pl.ds(s * 624, 624)], rg_sh.at[pl.ds(s * 624, 624)])
        pltpu.sync_copy(z_h.at[pl.ds(s * 312, 312)], rp_sh.at[pl.ds(s * 312, 312)])
        @pl.when(s == 15)
        def _():
            pltpu.sync_copy(z_h.at[pl.ds(9984, 16)], rg_sh.at[pl.ds(9984, 16)])
            pltpu.sync_copy(z_h.at[pl.ds(4992, 8)], rp_sh.at[pl.ds(4992, 8)])
        plsc.subcore_barrier()

        def run_edges(src_h, dst_h, ew_h, tab_h, acc_sh, nbatch_total):
            nb, base = _edge_partition(wid, nbatch_total)

            def body(b, carry):
                off = (base + b) * 64
                pltpu.sync_copy(src_h.at[pl.ds(off, 64)], idxs.at[0])
                pltpu.sync_copy(dst_h.at[pl.ds(off, 64)], idxd.at[0])
                pltpu.sync_copy(ew_h.at[pl.ds(off, 64)], ewb.at[0])
                pltpu.async_copy(tab_h.at[idxs.at[0]], rowsb, sem).wait()

                def sbody(gi, cc):
                    ev = ewb[0, pl.ds(gi * 16, 16)]
                    for kk in range(16):
                        e = gi * 16 + kk
                        bc = jnp.full((16,), ev[kk], F32)
                        for cg in range(4):
                            v = rowsb[e, pl.ds(cg * 16, 16)]
                            rowsb[e, pl.ds(cg * 16, 16)] = v * bc
                    return cc

                lax.fori_loop(0, 4, sbody, 0)
                pltpu.sync_copy(rowsb, acc_sh.at[idxd.at[0]], add=True)
                return carry

            lax.fori_loop(0, nb, body, 0)

        run_edges(srcg_h, dstg_h, ewg_h, pe2_h, rg_sh, ECO // 64)
        run_edges(srcp_h, dstp_h, ewp_h, pt2_h, rp_sh, EGO // 64)
        plsc.subcore_barrier()

        pltpu.sync_copy(rg_sh.at[pl.ds(s * 624, 624)], outg_h.at[pl.ds(c * NG + s * 624, 624)])
        pltpu.sync_copy(rp_sh.at[pl.ds(s * 312, 312)], outp_h.at[pl.ds(c * NP + s * 312, 312)])
        @pl.when(s == 15)
        def _():
            pltpu.sync_copy(rg_sh.at[pl.ds(9984, 16)], outg_h.at[pl.ds(c * NG + 9984, 16)])
            pltpu.sync_copy(rp_sh.at[pl.ds(4992, 8)], outp_h.at[pl.ds(c * NP + 4992, 8)])

    return k(src_g, dst_g, ew_g, pe2, src_p, dst_p, ew_p, pt2, zrows)


def _renorm_in(t):
    n2 = jnp.sum(t * t, axis=1, keepdims=True)
    scale = jnp.where(n2 > 1.0, lax.rsqrt(n2), 1.0)
    return t * scale


def _tc_norm(ge, pe, pt, bng, bnb):
    def body(ge_r, pe_r, pt_r, bng_r, bnb_r, base_o, pen_o, ptn_o):
        gen = _renorm_in(ge_r[...])
        mu = jnp.mean(gen, axis=0, keepdims=True)
        xc = gen - mu
        var = jnp.mean(xc * xc, axis=0, keepdims=True)
        base_o[...] = jnp.maximum(xc * lax.rsqrt(var + EPS) * bng_r[...] + bnb_r[...], 0.0)
        pen_o[...] = _renorm_in(pe_r[...])
        ptn_o[...] = _renorm_in(pt_r[...])

    return pl.pallas_call(
        body,
        out_shape=(jax.ShapeDtypeStruct((NG, H), F32),
                   jax.ShapeDtypeStruct((NG, H), F32),
                   jax.ShapeDtypeStruct((NP, H), F32)),
    )(ge, pe, pt, bng, bnb)


def _tc_scale(degg, degp, pen, ptn):
    def body(degg_r, degp_r, pen_r, ptn_r, pe2_o, pt2_o, dinv_o, dinvp_o):
        ones = jnp.ones((128, 128), F32)
        degb = jnp.dot(degg_r[0] + degg_r[1], ones, preferred_element_type=F32)
        degpb = jnp.dot(degp_r[0] + degp_r[1], ones, preferred_element_type=F32)
        dinv = lax.rsqrt(degb + 1.0)
        dinvp = lax.rsqrt(degpb + 1.0)
        dinv_o[...] = dinv
        dinvp_o[...] = dinvp
        pe2_o[...] = jnp.concatenate(
            [pen_r[...] * dinv[:, :H], jnp.zeros((NG, 128 - H), F32)], axis=1)
        pt2_o[...] = jnp.concatenate(
            [ptn_r[...] * dinvp[:, :H], jnp.zeros((NP, 128 - H), F32)], axis=1)

    return pl.pallas_call(
        body,
        out_shape=(jax.ShapeDtypeStruct((NG, 128), F32),
                   jax.ShapeDtypeStruct((NP, 128), F32),
                   jax.ShapeDtypeStruct((NG, 128), F32),
                   jax.ShapeDtypeStruct((NP, 128), F32)),
    )(degg, degp, pen, ptn)


def _wbn(hA, hB, g, b):
    s = (jnp.sum(hA, axis=0, keepdims=True) + 15.0 * jnp.sum(hB, axis=0, keepdims=True)) / NROWS
    s2 = (jnp.sum(hA * hA, axis=0, keepdims=True)
          + 15.0 * jnp.sum(hB * hB, axis=0, keepdims=True)) / NROWS
    var = s2 - s * s
    sc = g * lax.rsqrt(var + EPS)
    off = b - s * sc
    return hA * sc + off, hB * sc + off


def _bn16(x, g, b):
    mu = jnp.mean(x, axis=0, keepdims=True)
    xc = x - mu
    var = jnp.mean(xc * xc, axis=0, keepdims=True)
    return xc * lax.rsqrt(var + EPS) * g + b


def _tc_mid(rg, rp, pen, ptn, dinv, dinvp, base, pidx, pp):
    def body(rg_r, rp_r, pen_r, ptn_r, dinv_r, dinvp_r, base_r, pidx_r,
             sgw_r, sgb_r, ew1_r, eb1_r, eg1_r, ebb1_r, ew2_r, eb2_r, eg2_r, ebb2_r,
             spw_r, spb_r, pw1_r, pb1_r, pg1_r, pbb1_r, pw2_r, pb2_r, pg2_r, pbb2_r,
             pbg_r, pbb_r, ct_o, u_o):
        dinv = dinv_r[...][:, :H]
        pen = pen_r[...]
        r = rg_r[0, :, :H] + rg_r[1, :, :H]
        aggA = dinv * dinv * pen + dinv * r
        posA = jnp.dot(aggA, sgw_r[...], preferred_element_type=F32) + sgb_r[...]
        posB = jnp.dot(pen, sgw_r[...], preferred_element_type=F32) + sgb_r[...]
        base = base_r[...]
        A0 = base + 0.2 * posA
        B0 = base + 0.2 * posB
        hA = jnp.dot(A0, ew1_r[...], preferred_element_type=F32) + eb1_r[...]
        hB = jnp.dot(B0, ew1_r[...], preferred_element_type=F32) + eb1_r[...]
        hA, hB = _wbn(hA, hB, eg1_r[...], ebb1_r[...])
        hA = jnp.maximum(hA, 0.0)
        hB = jnp.maximum(hB, 0.0)
        hA = jnp.dot(hA, ew2_r[...], preferred_element_type=F32) + eb2_r[...]
        hB = jnp.dot(hB, ew2_r[...], preferred_element_type=F32) + eb2_r[...]
        A1, B1 = _wbn(hA, hB, eg2_r[...], ebb2_r[...])

        dinvp = dinvp_r[...][:, :H]
        ptn = ptn_r[...]
        rp = rp_r[0, :, :H] + rp_r[1, :, :H]
        aggP = dinvp * dinvp * ptn + dinvp * rp
        pg = jnp.dot(aggP, spw_r[...], preferred_element_type=F32) + spb_r[...]
        iot = lax.broadcasted_iota(jnp.int32, (NGR, NP), 1)
        pid = pidx_r[...]
        oh = ((iot == pid[:, 0:1]).astype(F32) + (iot == pid[:, 1:2]).astype(F32))
        track = jnp.dot(oh, pg, preferred_element_type=F32)
        t = jnp.dot(track, pw1_r[...], preferred_element_type=F32) + pb1_r[...]
        t = _bn16(t, pg1_r[...], pbb1_r[...])
        t = jnp.maximum(t, 0.0)
        t = jnp.dot(t, pw2_r[...], preferred_element_type=F32) + pb2_r[...]
        t = _bn16(t, pg2_r[...], pbb2_r[...])

        sA = jnp.sum(A1, axis=0, keepdims=True)
        sB = jnp.sum(B1, axis=0, keepdims=True)
        qA = jnp.sum(A1 * A1, axis=0, keepdims=True)
        qB = jnp.sum(B1 * B1, axis=0, keepdims=True)
        st = jnp.sum(t, axis=0, keepdims=True)
        t0 = t[0:1, :]
        st1 = st - t0
        q_t1 = jnp.sum(t * t, axis=0, keepdims=True) - t0 * t0
        S = (sA + 15.0 * sB + NG * st) / NROWS
        S2 = (qA + 2.0 * t0 * sA + NG * t0 * t0
              + 15.0 * qB + 2.0 * st1 * sB + NG * q_t1) / NROWS
        var = S2 - S * S
        s3 = pbg_r[...] * lax.rsqrt(var + EPS)
        off3 = pbb_r[...] - S * s3
        ct_o[0] = A1 * s3
        ct_o[1] = B1 * s3
        u_o[...] = t * s3 + off3

    return pl.pallas_call(
        body,
        out_shape=(jax.ShapeDtypeStruct((2, NG, H), F32),
                   jax.ShapeDtypeStruct((NGR, H), F32)),
    )(rg, rp, pen, ptn, dinv, dinvp, base, pidx,
      pp['sg_gene_W'], pp['sg_gene_b'].reshape(1, H),
      pp['etv2_W1'], pp['etv2_b1'].reshape(1, H), pp['etv2_g1'].reshape(1, H),
      pp['etv2_bb1'].reshape(1, H), pp['etv2_W2'], pp['etv2_b2'].reshape(1, H),
      pp['etv2_g2'].reshape(1, H), pp['etv2_bb2'].reshape(1, H),
      pp['sg_pert_W'], pp['sg_pert_b'].reshape(1, H),
      pp['pfuse_W1'], pp['pfuse_b1'].reshape(1, H), pp['pfuse_g1'].reshape(1, H),
      pp['pfuse_bb1'].reshape(1, H), pp['pfuse_W2'], pp['pfuse_b2'].reshape(1, H),
      pp['pfuse_g2'].reshape(1, H), pp['pfuse_bb2'].reshape(1, H),
      pp['bn_pb_g'].reshape(1, H), pp['bn_pb_b'].reshape(1, H))


def _first_step():
    return (pl.program_id(0) == 0) & (pl.program_id(1) == 0)


def _tc_stats1(ct, u):
    def body(ct_r, u_r, g_o, sr_o):
        r = jnp.maximum(ct_r[0] + u_r[0], 0.0)

        @pl.when(_first_step())
        def _():
            g_o[...] = jnp.zeros_like(g_o)
            sr_o[...] = jnp.zeros_like(sr_o)

        g_o[...] += lax.dot_general(r, r, (((0,), (0,)), ((), ())),
                                    preferred_element_type=F32)
        sr_o[...] += jnp.sum(r, axis=0, keepdims=True)

    return pl.pallas_call(
        body,
        grid=(NJ, NGR),
        in_specs=[
            pl.BlockSpec((1, TJ, H), lambda j, g: (jnp.minimum(g, 1), j, 0)),
            pl.BlockSpec((1, 1, H), lambda j, g: (g, 0, 0)),
        ],
        out_specs=[
            pl.BlockSpec((H, H), lambda j, g: (0, 0)),
            pl.BlockSpec((1, H), lambda j, g: (0, 0)),
        ],
        out_shape=(jax.ShapeDtypeStruct((H, H), F32),
                   jax.ShapeDtypeStruct((1, H), F32)),
    )(ct, u)


def _tc_h2(ct, u, w1, b1, s1, o1, w2, b2):
    def body(ct_r, u_r, w1_r, b1_r, s1_r, o1_r, w2_r, b2_r, h2_o, st_o):
        r = jnp.maximum(ct_r[0] + u_r[0], 0.0)
        h1 = jnp.dot(r, w1_r[...], preferred_element_type=F32) + b1_r[...]
        q = jnp.maximum(h1 * s1_r[...] + o1_r[...], 0.0)
        h2 = jnp.dot(q, w2_r[...], preferred_element_type=F32) + b2_r[...]
        h2_o[0] = h2

        @pl.when(_first_step())
        def _():
            st_o[...] = jnp.zeros_like(st_o)

        st_o[0, :] += jnp.sum(h2, axis=0)
        st_o[1, :] += jnp.sum(h2 * h2, axis=0)

    return pl.pallas_call(
        body,
        grid=(NJ, NGR),
        in_specs=[
            pl.BlockSpec((1, TJ, H), lambda j, g: (jnp.minimum(g, 1), j, 0)),
            pl.BlockSpec((1, 1, H), lambda j, g: (g, 0, 0)),
            pl.BlockSpec((H, 2 * H), lambda j, g: (0, 0)),
            pl.BlockSpec((1, 2 * H), lambda j, g: (0, 0)),
            pl.BlockSpec((1, 2 * H), lambda j, g: (0, 0)),
            pl.BlockSpec((1, 2 * H), lambda j, g: (0, 0)),
            pl.BlockSpec((2 * H, H), lambda j, g: (0, 0)),
            pl.BlockSpec((1, H), lambda j, g: (0, 0)),
        ],
        out_specs=[
            pl.BlockSpec((1, TJ, H), lambda j, g: (g, j, 0)),
            pl.BlockSpec((2, H), lambda j, g: (0, 0)),
        ],
        out_shape=(jax.ShapeDtypeStruct((NGR, NG, H), F32),
                   jax.ShapeDtypeStruct((2, H), F32)),
    )(ct, u, w1, b1, s1, o1, w2, b2)


def _tc_wdot(h2, v, s2, o2):
    def body(h2_r, v_r, s2_r, o2_r, w_o):
        out = h2_r[0] * s2_r[...] + o2_r[...]
        w_o[0, 0, 0, :] = jnp.sum(out * v_r[...], axis=1)

    return pl.pallas_call(
        body,
        grid=(NJ, NGR),
        in_specs=[
            pl.BlockSpec((1, TJ, H), lambda j, g: (g, j, 0)),
            pl.BlockSpec((TJ, H), lambda j, g: (j, 0)),
            pl.BlockSpec((1, H), lambda j, g: (0, 0)),
            pl.BlockSpec((1, H), lambda j, g: (0, 0)),
        ],
        out_specs=pl.BlockSpec((1, 1, 1, TJ), lambda j, g: (g, j, 0, 0)),
        out_shape=jax.ShapeDtypeStruct((NGR, NJ, 1, TJ), F32),
    )(h2, v, s2, o2)


def _tc_final(w, xr, b1v, cw1, cb1, cg1, cbb1, cw2, cb2, cg2, cbb2, w2c0, w2g, b2v):
    def body(w_r, xr_r, b1v_r, cw1_r, cb1_r, cg1_r, cbb1_r, cw2_r, cb2_r,
             cg2_r, cbb2_r, w2c0_r, w2g_r, b2v_r, out_o):
        outs = w_r[...] + b1v_r[...]
        t = jnp.dot(outs, cw1_r[...], preferred_element_type=F32) + cb1_r[...]
        t = _bn16(t, cg1_r[...], cbb1_r[...])
        t = jnp.maximum(t, 0.0)
        t = jnp.dot(t, cw2_r[...], preferred_element_type=F32) + cb2_r[...]
        cge = _bn16(t, cg2_r[...], cbb2_r[...])
        out_o[...] = (outs * w2c0_r[...]
                      + jnp.dot(cge, w2g_r[...], preferred_element_type=F32)
                      + b2v_r[...] + xr_r[...])

    return pl.pallas_call(
        body,
        out_shape=jax.ShapeDtypeStruct((NGR, NG), F32),
    )(w, xr, b1v, cw1, cb1, cg1, cbb1, cw2, cb2, cg2, cbb2, w2c0, w2g, b2v)


def kernel(x, pert_idx, batch, G_coexpress, G_coexpress_weight, G_go, G_go_weight, params):
    p = params
    src_g = G_coexpress[0].astype(jnp.int32)
    dst_g = G_coexpress[1].astype(jnp.int32)
    src_p = G_go[0].astype(jnp.int32)
    dst_p = G_go[1].astype(jnp.int32)
    ew_g = G_coexpress_weight
    ew_p = G_go_weight

    zrows = jnp.zeros((NG, 128), F32)

    degg_f, degp_f = _sc_deg(dst_g, ew_g, dst_p, ew_p, zrows)
    degg = degg_f.reshape(2, NG, 128)
    degp = degp_f.reshape(2, NP, 128)

    base, pen, ptn = _tc_norm(
        p['gene_emb'], p['pos_emb'], p['pert_emb'],
        p['bn_emb_g'].reshape(1, H), p['bn_emb_b'].reshape(1, H))
    pe2, pt2, dinv, dinvp = _tc_scale(degg, degp, pen, ptn)

    rg_f, rp_f = _sc_rows(src_g, dst_g, ew_g, pe2, src_p, dst_p, ew_p, pt2, zrows)
    rg = rg_f.reshape(2, NG, 128)
    rp = rp_f.reshape(2, NP, 128)

    ct, u = _tc_mid(rg, rp, pen, ptn, dinv, dinvp, base,
                    pert_idx.astype(jnp.int32), p)

    w1 = p['rec_W1']
    b1 = p['rec_b1'].reshape(1, 2 * H)
    u3 = u.reshape(NGR, 1, H)
    g_cov, sr = _tc_stats1(ct, u3)
    sr1 = sr[0]
    mu1 = (sr1 @ w1) / NROWS + b1[0]
    gw = g_cov @ w1
    e2 = (jnp.sum(w1 * gw, axis=0) + 2.0 * b1[0] * (sr1 @ w1) + NROWS * b1[0] ** 2) / NROWS
    var1 = e2 - mu1 * mu1
    s1 = p['rec_g1'] * lax.rsqrt(var1 + EPS)
    o1 = p['rec_bb1'] - mu1 * s1

    h2, st2 = _tc_h2(ct, u3, w1, b1, s1.reshape(1, 2 * H), o1.reshape(1, 2 * H),
                     p['rec_W2'], p['rec_b2'].reshape(1, H))
    mu2 = st2[0] / NROWS
    var2 = st2[1] / NROWS - mu2 * mu2
    s2 = p['rec_g2'] * lax.rsqrt(var2 + EPS)
    o2 = p['rec_b2'] * 0.0 + (p['rec_bb2'] - mu2 * s2)

    wmat = _tc_wdot(h2, p['indv_w1'][:, :, 0], s2.reshape(1, H), o2.reshape(1, H))
    wmat = wmat.reshape(NGR, NG)

    out2 = _tc_final(
        wmat, x.reshape(NGR, NG), p['indv_b1'][:, 0].reshape(1, NG),
        p['cross_W1'], p['cross_b1'].reshape(1, H), p['cross_g1'].reshape(1, H),
        p['cross_bb1'].reshape(1, H), p['cross_W2'], p['cross_b2'].reshape(1, H),
        p['cross_g2'].reshape(1, H), p['cross_bb2'].reshape(1, H),
        p['indv_w2'][0, :, 0].reshape(1, NG), p['indv_w2'][0, :, 1:].T,
        p['indv_b2'][0].reshape(1, NG))
    return out2
